# baseline (speedup 1.0000x reference)
import jax
import jax.numpy as jnp
from jax.experimental import pallas as pl
from jax.experimental.pallas import tpu as pltpu

_F32 = jnp.float32
_BF16 = jnp.bfloat16


def _ru(v, m):
    return ((v + m - 1) // m) * m


def _proj_kernel(h_ref, hu_ref, hd_ref, x_ref, xu_ref, xd_ref,
                 wia_ref, wj_ref, bj_ref, ti_ref, tj_ref):
    H = x_ref.shape[1]
    tP = h_ref.shape[0]
    Fj = wj_ref.shape[0] // 2
    zi = jnp.dot(h_ref[...], wia_ref[...], preferred_element_type=_F32)
    ti_ref[0, 0:2 * tP:2, :] = zi[:, :H]
    ti_ref[0, 1:2 * tP:2, :] = x_ref[...]
    ti_ref[1, 0:2 * tP:2, :] = zi[:, H:]
    ti_ref[1, 1:2 * tP:2, :] = x_ref[...]
    tj_ref[0, 0:2 * tP:2, :] = (jnp.dot(hu_ref[...], wj_ref[0:Fj, :],
                                        preferred_element_type=_F32)
                                + bj_ref[:, :H])
    tj_ref[0, 1:2 * tP:2, :] = xu_ref[...]
    tj_ref[1, 0:2 * tP:2, :] = (jnp.dot(hd_ref[...], wj_ref[Fj:, :],
                                        preferred_element_type=_F32)
                                + bj_ref[:, H:])
    tj_ref[1, 1:2 * tP:2, :] = xd_ref[...]


def _make_edge_kernel(tE):
    S = tE + 1

    def _gather(gi_ref, gj_ref, ti_ref, tj_ref, tile_i, tile_j):
        for mi in range(tE):
            i2 = pl.multiple_of(gi_ref[0, mi], 2)
            tile_i[mi:mi + 2 * S:S, :] = ti_ref[pl.ds(i2, 2), :]
            j2 = pl.multiple_of(gj_ref[0, mi], 2)
            tile_j[mi:mi + 2 * S:S, :] = tj_ref[pl.ds(j2, 2), :]

    def _compute(tile_i, tile_j, siv_ref, wm_ref, bv_ref, oh_ref, out_ref):
        H = bv_ref.shape[1]
        n_rows = out_ref.shape[0]
        w1x = bv_ref[0:1, :]
        b2 = bv_ref[1:2, :]
        cb1 = bv_ref[2:3, :]
        cb2 = bv_ref[3:4, :]

        ze = tile_i[pl.ds(0, tE), :] + tile_j[pl.ds(0, tE), :]
        xf = tile_i[pl.ds(S, tE), :] - tile_j[pl.ds(S, tE), :]

        x_msg = jnp.sum(xf * xf, axis=-1, keepdims=True)
        z1 = ze + x_msg * w1x
        m_e = (jnp.dot(jnp.maximum(z1, 0.0).astype(_BF16),
                       wm_ref[0:H, :], preferred_element_type=_F32) + b2)

        s1 = jnp.maximum(
            jnp.dot(m_e.astype(_BF16), wm_ref[H:2 * H, :],
                    preferred_element_type=_F32) + cb1, 0.0)
        gate = jax.nn.sigmoid(
            jnp.dot(s1.astype(_BF16), wm_ref[2 * H:3 * H, :],
                    preferred_element_type=_F32) + cb2)
        xs = xf * gate

        payload = jnp.concatenate(
            [m_e.astype(_BF16), xs.astype(_BF16)], axis=-1)
        rows = jax.lax.broadcasted_iota(jnp.int32, (n_rows, tE), 0)
        oh_ref[...] = (rows == siv_ref[...]).astype(_BF16)
        out_ref[...] = out_ref[...] + jnp.dot(
            oh_ref[...], payload, preferred_element_type=_F32)

    def _edge_kernel(giA_ref, gjA_ref, giB_ref, gjB_ref, sivA_ref, sivB_ref,
                     ti_ref, tj_ref, wm_ref, bv_ref, out_ref,
                     tiA, tjA, tiB, tjB, ohA, ohB):
        s = pl.program_id(1)

        @pl.when(s == 0)
        def _():
            out_ref[...] = jnp.zeros_like(out_ref)
            tiB[...] = jnp.zeros_like(tiB)
            tjB[...] = jnp.zeros_like(tjB)

        _gather(giA_ref, gjA_ref, ti_ref, tj_ref, tiA, tjA)
        _compute(tiB, tjB, sivA_ref, wm_ref, bv_ref, ohA, out_ref)
        _gather(giB_ref, gjB_ref, ti_ref, tj_ref, tiB, tjB)
        _compute(tiA, tjA, sivB_ref, wm_ref, bv_ref, ohB, out_ref)

    return _edge_kernel


def _cell_kernel(cw_ref, cin_ref, agg_ref, w1_ref, b1_ref, w2_ref, b2_ref,
                 out_ref):
    H = b1_ref.shape[1]
    Fp = w1_ref.shape[0] - 2 * H
    cin = cin_ref[...]
    h_p = cin[:, :Fp]
    x_p = cin[:, Fp:]
    Dp = x_p.shape[1]

    agg = agg_ref[...]
    m_up = agg[:, :H]
    xs_up = agg[:, H:H + Dp]
    m_dn = agg[:, H + Dp:2 * H + Dp]
    xs_dn = agg[:, 2 * H + Dp:]

    lhs = jnp.concatenate([h_p, m_up, m_dn], axis=-1).astype(_BF16)
    z1 = (jnp.dot(lhs, w1_ref[...], preferred_element_type=_F32)
          + b1_ref[...])
    h_upd = (jnp.dot(jnp.maximum(z1, 0.0).astype(_BF16), w2_ref[...],
                     preferred_element_type=_F32) + b2_ref[...])
    h_new = h_p + h_upd
    x_new = x_p + cw_ref[0] * xs_up + cw_ref[1] * xs_dn
    out_ref[...] = jnp.concatenate([h_new, x_new], axis=-1)


def kernel(p_up_W1, p_up_b1, p_up_W2, p_up_b2,
           p_dn_W1, p_dn_b1, p_dn_W2, p_dn_b2,
           p_cu_W1, p_cu_b1, p_cu_W2, p_cu_b2,
           p_cd_W1, p_cd_b1, p_cd_W2, p_cd_b2,
           p_cell_W1, p_cell_b1, p_cell_W2, p_cell_b2, p_cw,
           h, h_up, h_down, x, x_up, x_down,
           b_up_i, b_up_j, b_down_i, b_down_j):
    N, F = h.shape
    Nu, Fu = h_up.shape
    Nd, Fd = h_down.shape
    D = x.shape[1]
    H = p_up_b1.shape[1]
    O = p_cell_b2.shape[1]

    Fp = _ru(F, 128)
    Fm = max(_ru(Fu, 128), _ru(Fd, 128), Fp)
    Dp = _ru(D, 128)
    Op = _ru(O, 128)
    slab = H + Dp

    tP = min(512, _ru(max(N, Nu, Nd), 8))
    Np = _ru(max(N, Nu, Nd), tP)
    Eu, Ed = int(b_up_i.shape[0]), int(b_down_i.shape[0])
    tE = min(512, _ru(max(Eu, Ed, 1), 8))
    E_pad = _ru(max(Eu, Ed, 1), 2 * tE)
    n_e = E_pad // tE
    n_s = n_e // 2 + 1
    vmem_lim = 48 * 2**20

    wia = jnp.pad(jnp.concatenate([p_up_W1[:F], p_dn_W1[:F]], axis=1),
                  ((0, Fm - F), (0, 0))).astype(_BF16)
    wj = jnp.concatenate(
        [jnp.pad(p_up_W1[F:F + Fu], ((0, Fm - Fu), (0, 0))),
         jnp.pad(p_dn_W1[F:F + Fd], ((0, Fm - Fd), (0, 0)))],
        axis=0).astype(_BF16)
    bj = jnp.concatenate([p_up_b1, p_dn_b1], axis=1)

    wm = jnp.stack([
        jnp.concatenate([p_up_W2, p_cu_W1, jnp.tile(p_cu_W2, (1, H))], 0),
        jnp.concatenate([p_dn_W2, p_cd_W1, jnp.tile(p_cd_W2, (1, H))], 0),
    ]).astype(_BF16)
    bv = jnp.stack([
        jnp.concatenate([p_up_W1[F + Fu:F + Fu + 1], p_up_b2, p_cu_b1,
                         jnp.tile(p_cu_b2, (1, H))], 0),
        jnp.concatenate([p_dn_W1[F + Fd:F + Fd + 1], p_dn_b2, p_cd_b1,
                         jnp.tile(p_cd_b2, (1, H))], 0),
    ])

    hall = jnp.stack([jnp.pad(h, ((0, Np - N), (0, Fm - F))),
                      jnp.pad(h_up, ((0, Np - Nu), (0, Fm - Fu))),
                      jnp.pad(h_down, ((0, Np - Nd), (0, Fm - Fd)))]
                     ).astype(_BF16)
    xall = jnp.stack([jnp.pad(x, ((0, Np - N), (0, Dp - D))),
                      jnp.pad(x_up, ((0, Np - Nu), (0, Dp - D))),
                      jnp.pad(x_down, ((0, Np - Nd), (0, Dp - D)))]
                     ).astype(_F32)

    proj = pl.pallas_call(
        _proj_kernel,
        grid=(Np // tP,),
        in_specs=[
            pl.BlockSpec((None, tP, Fm), lambda i: (0, i, 0)),
            pl.BlockSpec((None, tP, Fm), lambda i: (1, i, 0)),
            pl.BlockSpec((None, tP, Fm), lambda i: (2, i, 0)),
            pl.BlockSpec((None, tP, Dp), lambda i: (0, i, 0)),
            pl.BlockSpec((None, tP, Dp), lambda i: (1, i, 0)),
            pl.BlockSpec((None, tP, Dp), lambda i: (2, i, 0)),
            pl.BlockSpec((Fm, 2 * H), lambda i: (0, 0)),
            pl.BlockSpec((2 * Fm, H), lambda i: (0, 0)),
            pl.BlockSpec((1, 2 * H), lambda i: (0, 0)),
        ],
        out_specs=[pl.BlockSpec((2, 2 * tP, Dp), lambda i: (0, i, 0))] * 2,
        out_shape=[jax.ShapeDtypeStruct((2, 2 * Np, Dp), _F32)] * 2,
        compiler_params=pltpu.CompilerParams(
            dimension_semantics=("parallel",), vmem_limit_bytes=vmem_lim),
    )
    ti_all, tj_all = proj(hall, hall, hall, xall, xall, xall, wia, wj, bj)

    L = E_pad + 2 * tE

    def gpad(idx, E):
        return jnp.pad(idx.astype(jnp.int32) * 2, (0, L - E))

    def spad(idx, E):
        return jnp.pad(idx.astype(jnp.int32), (tE, L - tE - E),
                       constant_values=Np)

    gi = jnp.stack([gpad(b_up_i, Eu), gpad(b_down_i, Ed)]).reshape(2, 1, L)
    gj = jnp.stack([gpad(b_up_j, Eu), gpad(b_down_j, Ed)]).reshape(2, 1, L)
    siv = jnp.stack([spad(b_up_i, Eu), spad(b_down_i, Ed)]).reshape(2, 1, L)

    smem = pltpu.MemorySpace.SMEM
    edge = pl.pallas_call(
        _make_edge_kernel(tE),
        grid=(2, n_s),
        in_specs=[
            pl.BlockSpec((None, 1, tE), lambda b, s: (b, 0, 2 * s),
                         memory_space=smem),
            pl.BlockSpec((None, 1, tE), lambda b, s: (b, 0, 2 * s),
                         memory_space=smem),
            pl.BlockSpec((None, 1, tE), lambda b, s: (b, 0, 2 * s + 1),
                         memory_space=smem),
            pl.BlockSpec((None, 1, tE), lambda b, s: (b, 0, 2 * s + 1),
                         memory_space=smem),
            pl.BlockSpec((None, 1, tE), lambda b, s: (b, 0, 2 * s)),
            pl.BlockSpec((None, 1, tE), lambda b, s: (b, 0, 2 * s + 1)),
            pl.BlockSpec((None, 2 * Np, Dp), lambda b, s: (b, 0, 0)),
            pl.BlockSpec((None, 2 * Np, Dp), lambda b, s: (b, 0, 0)),
            pl.BlockSpec((None, 3 * H, H), lambda b, s: (b, 0, 0)),
            pl.BlockSpec((None, 4, H), lambda b, s: (b, 0, 0)),
        ],
        out_specs=pl.BlockSpec((Np, slab), lambda b, s: (0, b)),
        out_shape=jax.ShapeDtypeStruct((Np, 2 * slab), _F32),
        scratch_shapes=[pltpu.VMEM(((tE + 1) * 2, Dp), _F32)] * 4
        + [pltpu.VMEM((Np, tE), _BF16)] * 2,
        compiler_params=pltpu.CompilerParams(
            dimension_semantics=("parallel", "arbitrary"),
            vmem_limit_bytes=vmem_lim),
    )
    agg = edge(gi, gj, gi, gj, siv, siv, ti_all, tj_all, wm, bv)

    cin = jnp.concatenate(
        [jnp.pad(h.astype(_F32), ((0, Np - N), (0, Fp - F))),
         jnp.pad(x.astype(_F32), ((0, Np - N), (0, Dp - D)))], axis=-1)

    w1c = jnp.concatenate(
        [jnp.pad(p_cell_W1[:F], ((0, Fp - F), (0, 0))),
         p_cell_W1[F:F + 2 * H]], axis=0).astype(_BF16)
    w2c = jnp.pad(p_cell_W2, ((0, 0), (0, Op - O))).astype(_BF16)
    b2c = jnp.pad(p_cell_b2, ((0, 0), (0, Op - O)))
    cw = p_cw.reshape(-1).astype(_F32)

    tN = min(512, _ru(N, 8))
    cell = pl.pallas_call(
        _cell_kernel,
        grid=(Np // tN,),
        in_specs=[
            pl.BlockSpec(memory_space=smem),
            pl.BlockSpec((tN, Fp + Dp), lambda i: (i, 0)),
            pl.BlockSpec((tN, 2 * slab), lambda i: (i, 0)),
            pl.BlockSpec((Fp + 2 * H, H), lambda i: (0, 0)),
            pl.BlockSpec((1, H), lambda i: (0, 0)),
            pl.BlockSpec((H, Op), lambda i: (0, 0)),
            pl.BlockSpec((1, Op), lambda i: (0, 0)),
        ],
        out_specs=pl.BlockSpec((tN, Op + Dp), lambda i: (i, 0)),
        out_shape=jax.ShapeDtypeStruct((Np, Op + Dp), _F32),
        compiler_params=pltpu.CompilerParams(
            dimension_semantics=("parallel",), vmem_limit_bytes=vmem_lim),
    )
    out = cell(cw, cin, agg, w1c, p_cell_b1, w2c, b2c)

    return out[:N, :O], out[:N, Op:Op + D]


# speedup vs baseline: 1.3175x; 1.3175x over previous
import jax
import jax.numpy as jnp
from jax.experimental import pallas as pl
from jax.experimental.pallas import tpu as pltpu

_F32 = jnp.float32
_BF16 = jnp.bfloat16


def _ru(v, m):
    return ((v + m - 1) // m) * m


def _proj_kernel(h_ref, hu_ref, hd_ref, x_ref, xu_ref, xd_ref,
                 wia_ref, wj_ref, bj_ref, ti_ref, tj_ref):
    H = x_ref.shape[1]
    tP = h_ref.shape[0]
    Fj = wj_ref.shape[0] // 2
    zi = jnp.dot(h_ref[...], wia_ref[...], preferred_element_type=_F32)
    ti_ref[0, 0:2 * tP:2, :] = zi[:, :H]
    ti_ref[0, 1:2 * tP:2, :] = x_ref[...]
    ti_ref[1, 0:2 * tP:2, :] = zi[:, H:]
    ti_ref[1, 1:2 * tP:2, :] = x_ref[...]
    tj_ref[0, 0:2 * tP:2, :] = (jnp.dot(hu_ref[...], wj_ref[0:Fj, :],
                                        preferred_element_type=_F32)
                                + bj_ref[:, :H])
    tj_ref[0, 1:2 * tP:2, :] = xu_ref[...]
    tj_ref[1, 0:2 * tP:2, :] = (jnp.dot(hd_ref[...], wj_ref[Fj:, :],
                                        preferred_element_type=_F32)
                                + bj_ref[:, H:])
    tj_ref[1, 1:2 * tP:2, :] = xd_ref[...]


def _make_edge_kernel(tE):
    S = tE + 1

    def _gather(gi_ref, gj_ref, ti_ref, tj_ref, tile_i, tile_j):
        for mi in range(tE):
            i2 = pl.multiple_of(gi_ref[0, mi], 2)
            tile_i[mi:mi + 2 * S:S, :] = ti_ref[pl.ds(i2, 2), :]
            j2 = pl.multiple_of(gj_ref[0, mi], 2)
            tile_j[mi:mi + 2 * S:S, :] = tj_ref[pl.ds(j2, 2), :]

    def _compute(tile_i, tile_j, siv_ref, wm_ref, bv_ref, oh_ref, out_ref):
        H = bv_ref.shape[1]
        n_rows = out_ref.shape[0]
        w1x = bv_ref[0:1, :]
        b2 = bv_ref[1:2, :]
        cb1 = bv_ref[2:3, :]
        cb2 = bv_ref[3:4, :]

        ze = tile_i[pl.ds(0, tE), :] + tile_j[pl.ds(0, tE), :]
        xf = tile_i[pl.ds(S, tE), :] - tile_j[pl.ds(S, tE), :]

        x_msg = jnp.sum(xf * xf, axis=-1, keepdims=True)
        z1 = ze + x_msg * w1x
        m_e = (jnp.dot(jnp.maximum(z1, 0.0).astype(_BF16),
                       wm_ref[0:H, :], preferred_element_type=_F32) + b2)

        s1 = jnp.maximum(
            jnp.dot(m_e.astype(_BF16), wm_ref[H:2 * H, :],
                    preferred_element_type=_F32) + cb1, 0.0)
        gate = jax.nn.sigmoid(
            jnp.dot(s1.astype(_BF16), wm_ref[2 * H:3 * H, :],
                    preferred_element_type=_F32) + cb2)
        xs = xf * gate

        payload = jnp.concatenate(
            [m_e.astype(_BF16), xs.astype(_BF16)], axis=-1)
        rows = jax.lax.broadcasted_iota(jnp.int32, (n_rows, tE), 0)
        oh_ref[...] = (rows == siv_ref[...]).astype(_BF16)
        out_ref[...] = out_ref[...] + jnp.dot(
            oh_ref[...], payload, preferred_element_type=_F32)

    def _edge_kernel(gi_ref, gj_ref, siv_ref, ti_ref, tj_ref,
                     wm_ref, bv_ref, out_ref, tile_i, tile_j, oh):
        e = pl.program_id(1)

        @pl.when(e == 0)
        def _():
            out_ref[...] = jnp.zeros_like(out_ref)

        _gather(gi_ref, gj_ref, ti_ref, tj_ref, tile_i, tile_j)
        _compute(tile_i, tile_j, siv_ref, wm_ref, bv_ref, oh, out_ref)

    return _edge_kernel


def _cell_kernel(cw_ref, cin_ref, agg_ref, w1_ref, b1_ref, w2_ref, b2_ref,
                 out_ref):
    H = b1_ref.shape[1]
    Fp = w1_ref.shape[0] - 2 * H
    cin = cin_ref[...]
    h_p = cin[:, :Fp]
    x_p = cin[:, Fp:]
    Dp = x_p.shape[1]

    agg = agg_ref[...]
    m_up = agg[:, :H]
    xs_up = agg[:, H:H + Dp]
    m_dn = agg[:, H + Dp:2 * H + Dp]
    xs_dn = agg[:, 2 * H + Dp:]

    lhs = jnp.concatenate([h_p, m_up, m_dn], axis=-1).astype(_BF16)
    z1 = (jnp.dot(lhs, w1_ref[...], preferred_element_type=_F32)
          + b1_ref[...])
    h_upd = (jnp.dot(jnp.maximum(z1, 0.0).astype(_BF16), w2_ref[...],
                     preferred_element_type=_F32) + b2_ref[...])
    h_new = h_p + h_upd
    x_new = x_p + cw_ref[0] * xs_up + cw_ref[1] * xs_dn
    out_ref[...] = jnp.concatenate([h_new, x_new], axis=-1)


def kernel(p_up_W1, p_up_b1, p_up_W2, p_up_b2,
           p_dn_W1, p_dn_b1, p_dn_W2, p_dn_b2,
           p_cu_W1, p_cu_b1, p_cu_W2, p_cu_b2,
           p_cd_W1, p_cd_b1, p_cd_W2, p_cd_b2,
           p_cell_W1, p_cell_b1, p_cell_W2, p_cell_b2, p_cw,
           h, h_up, h_down, x, x_up, x_down,
           b_up_i, b_up_j, b_down_i, b_down_j):
    N, F = h.shape
    Nu, Fu = h_up.shape
    Nd, Fd = h_down.shape
    D = x.shape[1]
    H = p_up_b1.shape[1]
    O = p_cell_b2.shape[1]

    Fp = _ru(F, 128)
    Fm = max(_ru(Fu, 128), _ru(Fd, 128), Fp)
    Dp = _ru(D, 128)
    Op = _ru(O, 128)
    slab = H + Dp

    tP = min(512, _ru(max(N, Nu, Nd), 8))
    Np = _ru(max(N, Nu, Nd), tP)
    Eu, Ed = int(b_up_i.shape[0]), int(b_down_i.shape[0])
    tE = min(1024, _ru(max(Eu, Ed, 1), 8))
    E_pad = _ru(max(Eu, Ed, 1), tE)
    n_e = E_pad // tE
    vmem_lim = 48 * 2**20

    wia = jnp.pad(jnp.concatenate([p_up_W1[:F], p_dn_W1[:F]], axis=1),
                  ((0, Fm - F), (0, 0))).astype(_BF16)
    wj = jnp.concatenate(
        [jnp.pad(p_up_W1[F:F + Fu], ((0, Fm - Fu), (0, 0))),
         jnp.pad(p_dn_W1[F:F + Fd], ((0, Fm - Fd), (0, 0)))],
        axis=0).astype(_BF16)
    bj = jnp.concatenate([p_up_b1, p_dn_b1], axis=1)

    wm = jnp.stack([
        jnp.concatenate([p_up_W2, p_cu_W1, jnp.tile(p_cu_W2, (1, H))], 0),
        jnp.concatenate([p_dn_W2, p_cd_W1, jnp.tile(p_cd_W2, (1, H))], 0),
    ]).astype(_BF16)
    bv = jnp.stack([
        jnp.concatenate([p_up_W1[F + Fu:F + Fu + 1], p_up_b2, p_cu_b1,
                         jnp.tile(p_cu_b2, (1, H))], 0),
        jnp.concatenate([p_dn_W1[F + Fd:F + Fd + 1], p_dn_b2, p_cd_b1,
                         jnp.tile(p_cd_b2, (1, H))], 0),
    ])

    hall = jnp.stack([jnp.pad(h, ((0, Np - N), (0, Fm - F))),
                      jnp.pad(h_up, ((0, Np - Nu), (0, Fm - Fu))),
                      jnp.pad(h_down, ((0, Np - Nd), (0, Fm - Fd)))]
                     ).astype(_BF16)
    xall = jnp.stack([jnp.pad(x, ((0, Np - N), (0, Dp - D))),
                      jnp.pad(x_up, ((0, Np - Nu), (0, Dp - D))),
                      jnp.pad(x_down, ((0, Np - Nd), (0, Dp - D)))]
                     ).astype(_F32)

    proj = pl.pallas_call(
        _proj_kernel,
        grid=(Np // tP,),
        in_specs=[
            pl.BlockSpec((None, tP, Fm), lambda i: (0, i, 0)),
            pl.BlockSpec((None, tP, Fm), lambda i: (1, i, 0)),
            pl.BlockSpec((None, tP, Fm), lambda i: (2, i, 0)),
            pl.BlockSpec((None, tP, Dp), lambda i: (0, i, 0)),
            pl.BlockSpec((None, tP, Dp), lambda i: (1, i, 0)),
            pl.BlockSpec((None, tP, Dp), lambda i: (2, i, 0)),
            pl.BlockSpec((Fm, 2 * H), lambda i: (0, 0)),
            pl.BlockSpec((2 * Fm, H), lambda i: (0, 0)),
            pl.BlockSpec((1, 2 * H), lambda i: (0, 0)),
        ],
        out_specs=[pl.BlockSpec((2, 2 * tP, Dp), lambda i: (0, i, 0))] * 2,
        out_shape=[jax.ShapeDtypeStruct((2, 2 * Np, Dp), _F32)] * 2,
        compiler_params=pltpu.CompilerParams(
            dimension_semantics=("parallel",), vmem_limit_bytes=vmem_lim),
    )
    ti_all, tj_all = proj(hall, hall, hall, xall, xall, xall, wia, wj, bj)

    def gpad(idx, E):
        return jnp.pad(idx.astype(jnp.int32) * 2, (0, E_pad - E))

    def spad(idx, E):
        return jnp.pad(idx.astype(jnp.int32), (0, E_pad - E),
                       constant_values=Np)

    gi = jnp.stack([gpad(b_up_i, Eu),
                    gpad(b_down_i, Ed)]).reshape(2, 1, E_pad)
    gj = jnp.stack([gpad(b_up_j, Eu),
                    gpad(b_down_j, Ed)]).reshape(2, 1, E_pad)
    siv = jnp.stack([spad(b_up_i, Eu),
                     spad(b_down_i, Ed)]).reshape(2, 1, E_pad)

    smem = pltpu.MemorySpace.SMEM
    edge = pl.pallas_call(
        _make_edge_kernel(tE),
        grid=(2, n_e),
        in_specs=[
            pl.BlockSpec((None, 1, tE), lambda b, e: (b, 0, e),
                         memory_space=smem),
            pl.BlockSpec((None, 1, tE), lambda b, e: (b, 0, e),
                         memory_space=smem),
            pl.BlockSpec((None, 1, tE), lambda b, e: (b, 0, e)),
            pl.BlockSpec((None, 2 * Np, Dp), lambda b, e: (b, 0, 0)),
            pl.BlockSpec((None, 2 * Np, Dp), lambda b, e: (b, 0, 0)),
            pl.BlockSpec((None, 3 * H, H), lambda b, e: (b, 0, 0)),
            pl.BlockSpec((None, 4, H), lambda b, e: (b, 0, 0)),
        ],
        out_specs=pl.BlockSpec((Np, slab), lambda b, e: (0, b)),
        out_shape=jax.ShapeDtypeStruct((Np, 2 * slab), _F32),
        scratch_shapes=[pltpu.VMEM(((tE + 1) * 2, Dp), _F32)] * 2
        + [pltpu.VMEM((Np, tE), _BF16)],
        compiler_params=pltpu.CompilerParams(
            dimension_semantics=("parallel", "arbitrary"),
            vmem_limit_bytes=vmem_lim),
    )
    agg = edge(gi, gj, siv, ti_all, tj_all, wm, bv)

    cin = jnp.concatenate(
        [jnp.pad(h.astype(_F32), ((0, Np - N), (0, Fp - F))),
         jnp.pad(x.astype(_F32), ((0, Np - N), (0, Dp - D)))], axis=-1)

    w1c = jnp.concatenate(
        [jnp.pad(p_cell_W1[:F], ((0, Fp - F), (0, 0))),
         p_cell_W1[F:F + 2 * H]], axis=0).astype(_BF16)
    w2c = jnp.pad(p_cell_W2, ((0, 0), (0, Op - O))).astype(_BF16)
    b2c = jnp.pad(p_cell_b2, ((0, 0), (0, Op - O)))
    cw = p_cw.reshape(-1).astype(_F32)

    tN = min(512, _ru(N, 8))
    cell = pl.pallas_call(
        _cell_kernel,
        grid=(Np // tN,),
        in_specs=[
            pl.BlockSpec(memory_space=smem),
            pl.BlockSpec((tN, Fp + Dp), lambda i: (i, 0)),
            pl.BlockSpec((tN, 2 * slab), lambda i: (i, 0)),
            pl.BlockSpec((Fp + 2 * H, H), lambda i: (0, 0)),
            pl.BlockSpec((1, H), lambda i: (0, 0)),
            pl.BlockSpec((H, Op), lambda i: (0, 0)),
            pl.BlockSpec((1, Op), lambda i: (0, 0)),
        ],
        out_specs=pl.BlockSpec((tN, Op + Dp), lambda i: (i, 0)),
        out_shape=jax.ShapeDtypeStruct((Np, Op + Dp), _F32),
        compiler_params=pltpu.CompilerParams(
            dimension_semantics=("parallel",), vmem_limit_bytes=vmem_lim),
    )
    out = cell(cw, cin, agg, w1c, p_cell_b1, w2c, b2c)

    return out[:N, :O], out[:N, Op:Op + D]


# speedup vs baseline: 1.3315x; 1.0106x over previous
import jax
import jax.numpy as jnp
from jax.experimental import pallas as pl
from jax.experimental.pallas import tpu as pltpu

_F32 = jnp.float32
_BF16 = jnp.bfloat16


def _ru(v, m):
    return ((v + m - 1) // m) * m


def _proj_kernel(h_ref, hu_ref, hd_ref, x_ref, xu_ref, xd_ref,
                 wia_ref, wj_ref, bj_ref, ti_ref, tj_ref):
    H = x_ref.shape[1]
    tP = h_ref.shape[0]
    Fj = wj_ref.shape[0] // 2
    zi = jnp.dot(h_ref[...], wia_ref[...], preferred_element_type=_F32)
    ti_ref[0, 0:2 * tP:2, :] = zi[:, :H]
    ti_ref[0, 1:2 * tP:2, :] = x_ref[...]
    ti_ref[1, 0:2 * tP:2, :] = zi[:, H:]
    ti_ref[1, 1:2 * tP:2, :] = x_ref[...]
    tj_ref[0, 0:2 * tP:2, :] = (jnp.dot(hu_ref[...], wj_ref[0:Fj, :],
                                        preferred_element_type=_F32)
                                + bj_ref[:, :H])
    tj_ref[0, 1:2 * tP:2, :] = xu_ref[...]
    tj_ref[1, 0:2 * tP:2, :] = (jnp.dot(hd_ref[...], wj_ref[Fj:, :],
                                        preferred_element_type=_F32)
                                + bj_ref[:, H:])
    tj_ref[1, 1:2 * tP:2, :] = xd_ref[...]


def _make_edge_kernel(tE):
    S = tE + 1

    def _gather(gi_ref, gj_ref, ti_ref, tj_ref, tile_i, tile_j):
        for mi in range(tE):
            i2 = pl.multiple_of(gi_ref[0, mi], 2)
            tile_i[mi:mi + 2 * S:S, :] = ti_ref[pl.ds(i2, 2), :]
            j2 = pl.multiple_of(gj_ref[0, mi], 2)
            tile_j[mi:mi + 2 * S:S, :] = tj_ref[pl.ds(j2, 2), :]

    def _compute(tile_i, tile_j, siv_ref, wm_ref, bv_ref, oh_ref, out_ref):
        H = bv_ref.shape[1]
        n_rows = out_ref.shape[0]
        w1x = bv_ref[0:1, :]
        b2 = bv_ref[1:2, :]
        cb1 = bv_ref[2:3, :]
        cb2 = bv_ref[3:4, :]

        ze = tile_i[pl.ds(0, tE), :] + tile_j[pl.ds(0, tE), :]
        xf = tile_i[pl.ds(S, tE), :] - tile_j[pl.ds(S, tE), :]

        x_msg = jnp.sum(xf * xf, axis=-1, keepdims=True)
        z1 = ze + x_msg * w1x
        m_e = (jnp.dot(jnp.maximum(z1, 0.0).astype(_BF16),
                       wm_ref[0:H, :], preferred_element_type=_F32) + b2)

        s1 = jnp.maximum(
            jnp.dot(m_e.astype(_BF16), wm_ref[H:2 * H, :],
                    preferred_element_type=_F32) + cb1, 0.0)
        gate = jax.nn.sigmoid(
            jnp.dot(s1.astype(_BF16), wm_ref[2 * H:3 * H, :],
                    preferred_element_type=_F32) + cb2)
        xs = xf * gate

        payload = jnp.concatenate(
            [m_e.astype(_BF16), xs.astype(_BF16)], axis=-1)
        rows = jax.lax.broadcasted_iota(jnp.int32, (n_rows, tE), 0)
        oh_ref[...] = (rows == siv_ref[...]).astype(_BF16)
        out_ref[...] = out_ref[...] + jnp.dot(
            oh_ref[...], payload, preferred_element_type=_F32)

    def _edge_kernel(gi_ref, gj_ref, siv_ref, ti_ref, tj_ref,
                     wm_ref, bv_ref, out_ref, tile_i, tile_j, oh):
        e = pl.program_id(1)

        @pl.when(e == 0)
        def _():
            out_ref[...] = jnp.zeros_like(out_ref)

        _gather(gi_ref, gj_ref, ti_ref, tj_ref, tile_i, tile_j)
        _compute(tile_i, tile_j, siv_ref, wm_ref, bv_ref, oh, out_ref)

    return _edge_kernel


def _cell_kernel(cw_ref, cin_ref, agg_ref, w1_ref, b1_ref, w2_ref, b2_ref,
                 out_ref):
    H = b1_ref.shape[1]
    Fp = w1_ref.shape[0] - 2 * H
    cin = cin_ref[...]
    h_p = cin[:, :Fp]
    x_p = cin[:, Fp:]
    Dp = x_p.shape[1]

    agg = agg_ref[...]
    m_up = agg[:, :H]
    xs_up = agg[:, H:H + Dp]
    m_dn = agg[:, H + Dp:2 * H + Dp]
    xs_dn = agg[:, 2 * H + Dp:]

    lhs = jnp.concatenate([h_p, m_up, m_dn], axis=-1).astype(_BF16)
    z1 = (jnp.dot(lhs, w1_ref[...], preferred_element_type=_F32)
          + b1_ref[...])
    h_upd = (jnp.dot(jnp.maximum(z1, 0.0).astype(_BF16), w2_ref[...],
                     preferred_element_type=_F32) + b2_ref[...])
    h_new = h_p + h_upd
    x_new = x_p + cw_ref[0] * xs_up + cw_ref[1] * xs_dn
    out_ref[...] = jnp.concatenate([h_new, x_new], axis=-1)


def kernel(p_up_W1, p_up_b1, p_up_W2, p_up_b2,
           p_dn_W1, p_dn_b1, p_dn_W2, p_dn_b2,
           p_cu_W1, p_cu_b1, p_cu_W2, p_cu_b2,
           p_cd_W1, p_cd_b1, p_cd_W2, p_cd_b2,
           p_cell_W1, p_cell_b1, p_cell_W2, p_cell_b2, p_cw,
           h, h_up, h_down, x, x_up, x_down,
           b_up_i, b_up_j, b_down_i, b_down_j):
    N, F = h.shape
    Nu, Fu = h_up.shape
    Nd, Fd = h_down.shape
    D = x.shape[1]
    H = p_up_b1.shape[1]
    O = p_cell_b2.shape[1]

    Fp = _ru(F, 128)
    Fm = max(_ru(Fu, 128), _ru(Fd, 128), Fp)
    Dp = _ru(D, 128)
    Op = _ru(O, 128)
    slab = H + Dp

    tP = min(512, _ru(max(N, Nu, Nd), 8))
    Np = _ru(max(N, Nu, Nd), tP)
    Eu, Ed = int(b_up_i.shape[0]), int(b_down_i.shape[0])
    tE = min(2048, _ru(max(Eu, Ed, 1), 8))
    E_pad = _ru(max(Eu, Ed, 1), tE)
    n_e = E_pad // tE
    vmem_lim = 48 * 2**20

    wia = jnp.pad(jnp.concatenate([p_up_W1[:F], p_dn_W1[:F]], axis=1),
                  ((0, Fm - F), (0, 0))).astype(_BF16)
    wj = jnp.concatenate(
        [jnp.pad(p_up_W1[F:F + Fu], ((0, Fm - Fu), (0, 0))),
         jnp.pad(p_dn_W1[F:F + Fd], ((0, Fm - Fd), (0, 0)))],
        axis=0).astype(_BF16)
    bj = jnp.concatenate([p_up_b1, p_dn_b1], axis=1)

    wm = jnp.stack([
        jnp.concatenate([p_up_W2, p_cu_W1, jnp.tile(p_cu_W2, (1, H))], 0),
        jnp.concatenate([p_dn_W2, p_cd_W1, jnp.tile(p_cd_W2, (1, H))], 0),
    ]).astype(_BF16)
    bv = jnp.stack([
        jnp.concatenate([p_up_W1[F + Fu:F + Fu + 1], p_up_b2, p_cu_b1,
                         jnp.tile(p_cu_b2, (1, H))], 0),
        jnp.concatenate([p_dn_W1[F + Fd:F + Fd + 1], p_dn_b2, p_cd_b1,
                         jnp.tile(p_cd_b2, (1, H))], 0),
    ])

    hall = jnp.stack([jnp.pad(h, ((0, Np - N), (0, Fm - F))),
                      jnp.pad(h_up, ((0, Np - Nu), (0, Fm - Fu))),
                      jnp.pad(h_down, ((0, Np - Nd), (0, Fm - Fd)))]
                     ).astype(_BF16)
    xall = jnp.stack([jnp.pad(x, ((0, Np - N), (0, Dp - D))),
                      jnp.pad(x_up, ((0, Np - Nu), (0, Dp - D))),
                      jnp.pad(x_down, ((0, Np - Nd), (0, Dp - D)))]
                     ).astype(_F32)

    proj = pl.pallas_call(
        _proj_kernel,
        grid=(Np // tP,),
        in_specs=[
            pl.BlockSpec((None, tP, Fm), lambda i: (0, i, 0)),
            pl.BlockSpec((None, tP, Fm), lambda i: (1, i, 0)),
            pl.BlockSpec((None, tP, Fm), lambda i: (2, i, 0)),
            pl.BlockSpec((None, tP, Dp), lambda i: (0, i, 0)),
            pl.BlockSpec((None, tP, Dp), lambda i: (1, i, 0)),
            pl.BlockSpec((None, tP, Dp), lambda i: (2, i, 0)),
            pl.BlockSpec((Fm, 2 * H), lambda i: (0, 0)),
            pl.BlockSpec((2 * Fm, H), lambda i: (0, 0)),
            pl.BlockSpec((1, 2 * H), lambda i: (0, 0)),
        ],
        out_specs=[pl.BlockSpec((2, 2 * tP, Dp), lambda i: (0, i, 0))] * 2,
        out_shape=[jax.ShapeDtypeStruct((2, 2 * Np, Dp), _F32)] * 2,
        compiler_params=pltpu.CompilerParams(
            dimension_semantics=("parallel",), vmem_limit_bytes=vmem_lim),
    )
    ti_all, tj_all = proj(hall, hall, hall, xall, xall, xall, wia, wj, bj)

    def gpad(idx, E):
        return jnp.pad(idx.astype(jnp.int32) * 2, (0, E_pad - E))

    def spad(idx, E):
        return jnp.pad(idx.astype(jnp.int32), (0, E_pad - E),
                       constant_values=Np)

    gi = jnp.stack([gpad(b_up_i, Eu),
                    gpad(b_down_i, Ed)]).reshape(2, 1, E_pad)
    gj = jnp.stack([gpad(b_up_j, Eu),
                    gpad(b_down_j, Ed)]).reshape(2, 1, E_pad)
    siv = jnp.stack([spad(b_up_i, Eu),
                     spad(b_down_i, Ed)]).reshape(2, 1, E_pad)

    smem = pltpu.MemorySpace.SMEM
    edge = pl.pallas_call(
        _make_edge_kernel(tE),
        grid=(2, n_e),
        in_specs=[
            pl.BlockSpec((None, 1, tE), lambda b, e: (b, 0, e),
                         memory_space=smem),
            pl.BlockSpec((None, 1, tE), lambda b, e: (b, 0, e),
                         memory_space=smem),
            pl.BlockSpec((None, 1, tE), lambda b, e: (b, 0, e)),
            pl.BlockSpec((None, 2 * Np, Dp), lambda b, e: (b, 0, 0)),
            pl.BlockSpec((None, 2 * Np, Dp), lambda b, e: (b, 0, 0)),
            pl.BlockSpec((None, 3 * H, H), lambda b, e: (b, 0, 0)),
            pl.BlockSpec((None, 4, H), lambda b, e: (b, 0, 0)),
        ],
        out_specs=pl.BlockSpec((Np, slab), lambda b, e: (0, b)),
        out_shape=jax.ShapeDtypeStruct((Np, 2 * slab), _F32),
        scratch_shapes=[pltpu.VMEM(((tE + 1) * 2, Dp), _F32)] * 2
        + [pltpu.VMEM((Np, tE), _BF16)],
        compiler_params=pltpu.CompilerParams(
            dimension_semantics=("parallel", "arbitrary"),
            vmem_limit_bytes=vmem_lim),
    )
    agg = edge(gi, gj, siv, ti_all, tj_all, wm, bv)

    cin = jnp.concatenate(
        [jnp.pad(h.astype(_F32), ((0, Np - N), (0, Fp - F))),
         jnp.pad(x.astype(_F32), ((0, Np - N), (0, Dp - D)))], axis=-1)

    w1c = jnp.concatenate(
        [jnp.pad(p_cell_W1[:F], ((0, Fp - F), (0, 0))),
         p_cell_W1[F:F + 2 * H]], axis=0).astype(_BF16)
    w2c = jnp.pad(p_cell_W2, ((0, 0), (0, Op - O))).astype(_BF16)
    b2c = jnp.pad(p_cell_b2, ((0, 0), (0, Op - O)))
    cw = p_cw.reshape(-1).astype(_F32)

    tN = min(512, _ru(N, 8))
    cell = pl.pallas_call(
        _cell_kernel,
        grid=(Np // tN,),
        in_specs=[
            pl.BlockSpec(memory_space=smem),
            pl.BlockSpec((tN, Fp + Dp), lambda i: (i, 0)),
            pl.BlockSpec((tN, 2 * slab), lambda i: (i, 0)),
            pl.BlockSpec((Fp + 2 * H, H), lambda i: (0, 0)),
            pl.BlockSpec((1, H), lambda i: (0, 0)),
            pl.BlockSpec((H, Op), lambda i: (0, 0)),
            pl.BlockSpec((1, Op), lambda i: (0, 0)),
        ],
        out_specs=pl.BlockSpec((tN, Op + Dp), lambda i: (i, 0)),
        out_shape=jax.ShapeDtypeStruct((Np, Op + Dp), _F32),
        compiler_params=pltpu.CompilerParams(
            dimension_semantics=("parallel",), vmem_limit_bytes=vmem_lim),
    )
    out = cell(cw, cin, agg, w1c, p_cell_b1, w2c, b2c)

    return out[:N, :O], out[:N, Op:Op + D]


# speedup vs baseline: 1.7778x; 1.3353x over previous
import jax
import jax.numpy as jnp
from jax.experimental import pallas as pl
from jax.experimental.pallas import tpu as pltpu

_F32 = jnp.float32
_BF16 = jnp.bfloat16


def _ru(v, m):
    return ((v + m - 1) // m) * m


def _proj_kernel(h_ref, hu_ref, hd_ref, x_ref, xu_ref, xd_ref,
                 wia_ref, wj_ref, bj_ref, ti_ref, tj_ref):
    H = x_ref.shape[1]
    tP = h_ref.shape[0]
    Fj = wj_ref.shape[0] // 2
    zi = jnp.dot(h_ref[...], wia_ref[...], preferred_element_type=_F32)
    ti_ref[0, 0:2 * tP:2, :] = zi[:, :H]
    ti_ref[0, 1:2 * tP:2, :] = x_ref[...]
    ti_ref[1, 0:2 * tP:2, :] = zi[:, H:]
    ti_ref[1, 1:2 * tP:2, :] = x_ref[...]
    tj_ref[0, 0:2 * tP:2, :] = (jnp.dot(hu_ref[...], wj_ref[0:Fj, :],
                                        preferred_element_type=_F32)
                                + bj_ref[:, :H])
    tj_ref[0, 1:2 * tP:2, :] = xu_ref[...]
    tj_ref[1, 0:2 * tP:2, :] = (jnp.dot(hd_ref[...], wj_ref[Fj:, :],
                                        preferred_element_type=_F32)
                                + bj_ref[:, H:])
    tj_ref[1, 1:2 * tP:2, :] = xd_ref[...]


def _make_edge_kernel(tE):
    S = tE + 1

    def _gather(gi_ref, gj_ref, ti_ref, tj_ref, tile_i, tile_j):
        for mi in range(tE):
            i2 = pl.multiple_of(gi_ref[0, mi], 2)
            tile_i[mi:mi + 2 * S:S, :] = ti_ref[pl.ds(i2, 2), :]
            j2 = pl.multiple_of(gj_ref[0, mi], 2)
            tile_j[mi:mi + 2 * S:S, :] = tj_ref[pl.ds(j2, 2), :]

    def _compute(tile_i, tile_j, siv_ref, wm_ref, bv_ref, out_ref, w0, w1):
        H = bv_ref.shape[1]
        n_rows = out_ref.shape[0]
        w1x = bv_ref[0:1, :]
        b2 = bv_ref[1:2, :]
        cb1 = bv_ref[2:3, :]
        cb2 = bv_ref[3:4, :]

        ze = tile_i[pl.ds(0, tE), :] + tile_j[pl.ds(0, tE), :]
        xf = tile_i[pl.ds(S, tE), :] - tile_j[pl.ds(S, tE), :]

        x_msg = jnp.sum(xf * xf, axis=-1, keepdims=True)
        z1 = ze + x_msg * w1x
        m_e = (jnp.dot(jnp.maximum(z1, 0.0).astype(_BF16),
                       wm_ref[0:H, :], preferred_element_type=_F32) + b2)

        s1 = jnp.maximum(
            jnp.dot(m_e.astype(_BF16), wm_ref[H:2 * H, :],
                    preferred_element_type=_F32) + cb1, 0.0)
        gate = jax.nn.sigmoid(
            jnp.dot(s1.astype(_BF16), wm_ref[2 * H:3 * H, :],
                    preferred_element_type=_F32) + cb2)
        xs = xf * gate

        payload = jnp.concatenate(
            [m_e.astype(_BF16), xs.astype(_BF16)], axis=-1)

        W = min(512, n_rows)
        siv = siv_ref[...]
        rows_w = jax.lax.broadcasted_iota(jnp.int32, (W, tE), 0)

        def _win(w, carry):
            base = w * W
            oneh = (rows_w == (siv - base)).astype(_BF16)
            out_ref[pl.ds(base, W), :] = (
                out_ref[pl.ds(base, W), :]
                + jnp.dot(oneh, payload, preferred_element_type=_F32))
            return carry

        jax.lax.fori_loop(w0, w1 + 1, _win, 0)

    def _edge_kernel(gi_ref, gj_ref, siv_ref, win_ref, ti_ref, tj_ref,
                     wm_ref, bv_ref, out_ref, tile_i, tile_j):
        e = pl.program_id(1)

        @pl.when(e == 0)
        def _():
            out_ref[...] = jnp.zeros_like(out_ref)

        _gather(gi_ref, gj_ref, ti_ref, tj_ref, tile_i, tile_j)
        _compute(tile_i, tile_j, siv_ref, wm_ref, bv_ref, out_ref,
                 win_ref[0, e], win_ref[1, e])

    return _edge_kernel


def _cell_kernel(cw_ref, cin_ref, agg_ref, w1_ref, b1_ref, w2_ref, b2_ref,
                 out_ref):
    H = b1_ref.shape[1]
    Fp = w1_ref.shape[0] - 2 * H
    cin = cin_ref[...]
    h_p = cin[:, :Fp]
    x_p = cin[:, Fp:]
    Dp = x_p.shape[1]

    agg = agg_ref[...]
    m_up = agg[:, :H]
    xs_up = agg[:, H:H + Dp]
    m_dn = agg[:, H + Dp:2 * H + Dp]
    xs_dn = agg[:, 2 * H + Dp:]

    lhs = jnp.concatenate([h_p, m_up, m_dn], axis=-1).astype(_BF16)
    z1 = (jnp.dot(lhs, w1_ref[...], preferred_element_type=_F32)
          + b1_ref[...])
    h_upd = (jnp.dot(jnp.maximum(z1, 0.0).astype(_BF16), w2_ref[...],
                     preferred_element_type=_F32) + b2_ref[...])
    h_new = h_p + h_upd
    x_new = x_p + cw_ref[0] * xs_up + cw_ref[1] * xs_dn
    out_ref[...] = jnp.concatenate([h_new, x_new], axis=-1)


def kernel(p_up_W1, p_up_b1, p_up_W2, p_up_b2,
           p_dn_W1, p_dn_b1, p_dn_W2, p_dn_b2,
           p_cu_W1, p_cu_b1, p_cu_W2, p_cu_b2,
           p_cd_W1, p_cd_b1, p_cd_W2, p_cd_b2,
           p_cell_W1, p_cell_b1, p_cell_W2, p_cell_b2, p_cw,
           h, h_up, h_down, x, x_up, x_down,
           b_up_i, b_up_j, b_down_i, b_down_j):
    N, F = h.shape
    Nu, Fu = h_up.shape
    Nd, Fd = h_down.shape
    D = x.shape[1]
    H = p_up_b1.shape[1]
    O = p_cell_b2.shape[1]

    Fp = _ru(F, 128)
    Fm = max(_ru(Fu, 128), _ru(Fd, 128), Fp)
    Dp = _ru(D, 128)
    Op = _ru(O, 128)
    slab = H + Dp

    tP = min(512, _ru(max(N, Nu, Nd), 8))
    Np = _ru(max(N, Nu, Nd), tP)
    Eu, Ed = int(b_up_i.shape[0]), int(b_down_i.shape[0])
    tE = min(2048, _ru(max(Eu, Ed, 1), 8))
    E_pad = _ru(max(Eu, Ed, 1), tE)
    n_e = E_pad // tE
    vmem_lim = 48 * 2**20

    wia = jnp.pad(jnp.concatenate([p_up_W1[:F], p_dn_W1[:F]], axis=1),
                  ((0, Fm - F), (0, 0))).astype(_BF16)
    wj = jnp.concatenate(
        [jnp.pad(p_up_W1[F:F + Fu], ((0, Fm - Fu), (0, 0))),
         jnp.pad(p_dn_W1[F:F + Fd], ((0, Fm - Fd), (0, 0)))],
        axis=0).astype(_BF16)
    bj = jnp.concatenate([p_up_b1, p_dn_b1], axis=1)

    wm = jnp.stack([
        jnp.concatenate([p_up_W2, p_cu_W1, jnp.tile(p_cu_W2, (1, H))], 0),
        jnp.concatenate([p_dn_W2, p_cd_W1, jnp.tile(p_cd_W2, (1, H))], 0),
    ]).astype(_BF16)
    bv = jnp.stack([
        jnp.concatenate([p_up_W1[F + Fu:F + Fu + 1], p_up_b2, p_cu_b1,
                         jnp.tile(p_cu_b2, (1, H))], 0),
        jnp.concatenate([p_dn_W1[F + Fd:F + Fd + 1], p_dn_b2, p_cd_b1,
                         jnp.tile(p_cd_b2, (1, H))], 0),
    ])

    hall = jnp.stack([jnp.pad(h, ((0, Np - N), (0, Fm - F))),
                      jnp.pad(h_up, ((0, Np - Nu), (0, Fm - Fu))),
                      jnp.pad(h_down, ((0, Np - Nd), (0, Fm - Fd)))]
                     ).astype(_BF16)
    xall = jnp.stack([jnp.pad(x, ((0, Np - N), (0, Dp - D))),
                      jnp.pad(x_up, ((0, Np - Nu), (0, Dp - D))),
                      jnp.pad(x_down, ((0, Np - Nd), (0, Dp - D)))]
                     ).astype(_F32)

    proj = pl.pallas_call(
        _proj_kernel,
        grid=(Np // tP,),
        in_specs=[
            pl.BlockSpec((None, tP, Fm), lambda i: (0, i, 0)),
            pl.BlockSpec((None, tP, Fm), lambda i: (1, i, 0)),
            pl.BlockSpec((None, tP, Fm), lambda i: (2, i, 0)),
            pl.BlockSpec((None, tP, Dp), lambda i: (0, i, 0)),
            pl.BlockSpec((None, tP, Dp), lambda i: (1, i, 0)),
            pl.BlockSpec((None, tP, Dp), lambda i: (2, i, 0)),
            pl.BlockSpec((Fm, 2 * H), lambda i: (0, 0)),
            pl.BlockSpec((2 * Fm, H), lambda i: (0, 0)),
            pl.BlockSpec((1, 2 * H), lambda i: (0, 0)),
        ],
        out_specs=[pl.BlockSpec((2, 2 * tP, Dp), lambda i: (0, i, 0))] * 2,
        out_shape=[jax.ShapeDtypeStruct((2, 2 * Np, Dp), _F32)] * 2,
        compiler_params=pltpu.CompilerParams(
            dimension_semantics=("parallel",), vmem_limit_bytes=vmem_lim),
    )
    ti_all, tj_all = proj(hall, hall, hall, xall, xall, xall, wia, wj, bj)

    sbi_u, sbj_u = jax.lax.sort_key_val(b_up_i.astype(jnp.int32),
                                        b_up_j.astype(jnp.int32))
    sbi_d, sbj_d = jax.lax.sort_key_val(b_down_i.astype(jnp.int32),
                                        b_down_j.astype(jnp.int32))

    def gpad(idx, E):
        return jnp.pad(idx * 2, (0, E_pad - E))

    def spad(idx, E):
        return jnp.pad(idx, (0, E_pad - E), constant_values=Np)

    gi = jnp.stack([gpad(sbi_u, Eu),
                    gpad(sbi_d, Ed)]).reshape(2, 1, E_pad)
    gj = jnp.stack([gpad(sbj_u, Eu),
                    gpad(sbj_d, Ed)]).reshape(2, 1, E_pad)
    siv = jnp.stack([spad(sbi_u, Eu),
                     spad(sbi_d, Ed)]).reshape(2, 1, E_pad)

    W = min(512, Np)
    keys = siv.reshape(2, E_pad)
    kc = jnp.minimum(keys, Np - 1)
    win = jnp.stack([kc[:, 0::tE] // W, kc[:, tE - 1::tE] // W], axis=1)

    smem = pltpu.MemorySpace.SMEM
    edge = pl.pallas_call(
        _make_edge_kernel(tE),
        grid=(2, n_e),
        in_specs=[
            pl.BlockSpec((None, 1, tE), lambda b, e: (b, 0, e),
                         memory_space=smem),
            pl.BlockSpec((None, 1, tE), lambda b, e: (b, 0, e),
                         memory_space=smem),
            pl.BlockSpec((None, 1, tE), lambda b, e: (b, 0, e)),
            pl.BlockSpec((None, 2, n_e), lambda b, e: (b, 0, 0),
                         memory_space=smem),
            pl.BlockSpec((None, 2 * Np, Dp), lambda b, e: (b, 0, 0)),
            pl.BlockSpec((None, 2 * Np, Dp), lambda b, e: (b, 0, 0)),
            pl.BlockSpec((None, 3 * H, H), lambda b, e: (b, 0, 0)),
            pl.BlockSpec((None, 4, H), lambda b, e: (b, 0, 0)),
        ],
        out_specs=pl.BlockSpec((Np, slab), lambda b, e: (0, b)),
        out_shape=jax.ShapeDtypeStruct((Np, 2 * slab), _F32),
        scratch_shapes=[pltpu.VMEM(((tE + 1) * 2, Dp), _F32)] * 2,
        compiler_params=pltpu.CompilerParams(
            dimension_semantics=("parallel", "arbitrary"),
            vmem_limit_bytes=vmem_lim),
    )
    agg = edge(gi, gj, siv, win, ti_all, tj_all, wm, bv)

    cin = jnp.concatenate(
        [jnp.pad(h.astype(_F32), ((0, Np - N), (0, Fp - F))),
         jnp.pad(x.astype(_F32), ((0, Np - N), (0, Dp - D)))], axis=-1)

    w1c = jnp.concatenate(
        [jnp.pad(p_cell_W1[:F], ((0, Fp - F), (0, 0))),
         p_cell_W1[F:F + 2 * H]], axis=0).astype(_BF16)
    w2c = jnp.pad(p_cell_W2, ((0, 0), (0, Op - O))).astype(_BF16)
    b2c = jnp.pad(p_cell_b2, ((0, 0), (0, Op - O)))
    cw = p_cw.reshape(-1).astype(_F32)

    tN = min(512, _ru(N, 8))
    cell = pl.pallas_call(
        _cell_kernel,
        grid=(Np // tN,),
        in_specs=[
            pl.BlockSpec(memory_space=smem),
            pl.BlockSpec((tN, Fp + Dp), lambda i: (i, 0)),
            pl.BlockSpec((tN, 2 * slab), lambda i: (i, 0)),
            pl.BlockSpec((Fp + 2 * H, H), lambda i: (0, 0)),
            pl.BlockSpec((1, H), lambda i: (0, 0)),
            pl.BlockSpec((H, Op), lambda i: (0, 0)),
            pl.BlockSpec((1, Op), lambda i: (0, 0)),
        ],
        out_specs=pl.BlockSpec((tN, Op + Dp), lambda i: (i, 0)),
        out_shape=jax.ShapeDtypeStruct((Np, Op + Dp), _F32),
        compiler_params=pltpu.CompilerParams(
            dimension_semantics=("parallel",), vmem_limit_bytes=vmem_lim),
    )
    out = cell(cw, cin, agg, w1c, p_cell_b1, w2c, b2c)

    return out[:N, :O], out[:N, Op:Op + D]


# speedup vs baseline: 1.7915x; 1.0077x over previous
import jax
import jax.numpy as jnp
from jax.experimental import pallas as pl
from jax.experimental.pallas import tpu as pltpu

_F32 = jnp.float32
_BF16 = jnp.bfloat16


def _ru(v, m):
    return ((v + m - 1) // m) * m


def _proj_kernel(h_ref, hu_ref, hd_ref, x_ref, xu_ref, xd_ref,
                 wia_ref, wj_ref, bj_ref, ti_ref, tj_ref):
    H = x_ref.shape[1]
    tP = h_ref.shape[0]
    Fj = wj_ref.shape[0] // 2
    zi = jnp.dot(h_ref[...], wia_ref[...], preferred_element_type=_F32)
    ti_ref[0, :, :H] = zi[:, :H].astype(_BF16)
    ti_ref[0, :, H:] = x_ref[...].astype(_BF16)
    ti_ref[1, :, :H] = zi[:, H:].astype(_BF16)
    ti_ref[1, :, H:] = x_ref[...].astype(_BF16)
    tj_ref[0, 0:2 * tP:2, :] = (jnp.dot(hu_ref[...], wj_ref[0:Fj, :],
                                        preferred_element_type=_F32)
                                + bj_ref[:, :H])
    tj_ref[0, 1:2 * tP:2, :] = xu_ref[...]
    tj_ref[1, 0:2 * tP:2, :] = (jnp.dot(hd_ref[...], wj_ref[Fj:, :],
                                        preferred_element_type=_F32)
                                + bj_ref[:, H:])
    tj_ref[1, 1:2 * tP:2, :] = xd_ref[...]


def _make_edge_kernel(tE):
    S = tE + 1

    def _gather(gj_ref, tj_ref, tile_j):
        for mi in range(tE):
            j2 = pl.multiple_of(gj_ref[0, mi], 2)
            tile_j[mi:mi + 2 * S:S, :] = tj_ref[pl.ds(j2, 2), :]

    def _compute(tile_j, siv_ref, ti_ref, wm_ref, bv_ref, out_ref, zx_ref,
                 w0, w1):
        H = bv_ref.shape[1]
        n_rows = out_ref.shape[0]
        w1x = bv_ref[0:1, :]
        b2 = bv_ref[1:2, :]
        cb1 = bv_ref[2:3, :]
        cb2 = bv_ref[3:4, :]

        W = min(512, n_rows)
        siv = siv_ref[...]
        rows_w = jax.lax.broadcasted_iota(jnp.int32, (W, tE), 0)

        zx_ref[...] = jnp.zeros_like(zx_ref)

        def _win1(w, carry):
            base = w * W
            oneh = (rows_w == (siv - base)).astype(_BF16)
            slab = ti_ref[pl.ds(base, W), :]
            zx_ref[...] = zx_ref[...] + jax.lax.dot_general(
                oneh, slab, (((0,), (0,)), ((), ())),
                preferred_element_type=_F32)
            return carry

        jax.lax.fori_loop(w0, w1 + 1, _win1, 0)

        zx = zx_ref[...]
        ze = zx[:, :H] + tile_j[pl.ds(0, tE), :]
        xf = zx[:, H:] - tile_j[pl.ds(S, tE), :]

        x_msg = jnp.sum(xf * xf, axis=-1, keepdims=True)
        z1 = ze + x_msg * w1x
        m_e = (jnp.dot(jnp.maximum(z1, 0.0).astype(_BF16),
                       wm_ref[0:H, :], preferred_element_type=_F32) + b2)

        s1 = jnp.maximum(
            jnp.dot(m_e.astype(_BF16), wm_ref[H:2 * H, :],
                    preferred_element_type=_F32) + cb1, 0.0)
        gate = jax.nn.sigmoid(
            jnp.dot(s1.astype(_BF16), wm_ref[2 * H:3 * H, :],
                    preferred_element_type=_F32) + cb2)
        xs = xf * gate

        payload = jnp.concatenate(
            [m_e.astype(_BF16), xs.astype(_BF16)], axis=-1)

        def _win(w, carry):
            base = w * W
            oneh = (rows_w == (siv - base)).astype(_BF16)
            out_ref[pl.ds(base, W), :] = (
                out_ref[pl.ds(base, W), :]
                + jnp.dot(oneh, payload, preferred_element_type=_F32))
            return carry

        jax.lax.fori_loop(w0, w1 + 1, _win, 0)

    def _edge_kernel(gj_ref, siv_ref, win_ref, ti_ref, tj_ref,
                     wm_ref, bv_ref, out_ref, tile_j, zx_ref):
        e = pl.program_id(1)

        @pl.when(e == 0)
        def _():
            out_ref[...] = jnp.zeros_like(out_ref)

        _gather(gj_ref, tj_ref, tile_j)
        _compute(tile_j, siv_ref, ti_ref, wm_ref, bv_ref, out_ref, zx_ref,
                 win_ref[0, e], win_ref[1, e])

    return _edge_kernel


def _cell_kernel(cw_ref, cin_ref, agg_ref, w1_ref, b1_ref, w2_ref, b2_ref,
                 out_ref):
    H = b1_ref.shape[1]
    Fp = w1_ref.shape[0] - 2 * H
    cin = cin_ref[...]
    h_p = cin[:, :Fp]
    x_p = cin[:, Fp:]
    Dp = x_p.shape[1]

    agg = agg_ref[...]
    m_up = agg[:, :H]
    xs_up = agg[:, H:H + Dp]
    m_dn = agg[:, H + Dp:2 * H + Dp]
    xs_dn = agg[:, 2 * H + Dp:]

    lhs = jnp.concatenate([h_p, m_up, m_dn], axis=-1).astype(_BF16)
    z1 = (jnp.dot(lhs, w1_ref[...], preferred_element_type=_F32)
          + b1_ref[...])
    h_upd = (jnp.dot(jnp.maximum(z1, 0.0).astype(_BF16), w2_ref[...],
                     preferred_element_type=_F32) + b2_ref[...])
    h_new = h_p + h_upd
    x_new = x_p + cw_ref[0] * xs_up + cw_ref[1] * xs_dn
    out_ref[...] = jnp.concatenate([h_new, x_new], axis=-1)


def kernel(p_up_W1, p_up_b1, p_up_W2, p_up_b2,
           p_dn_W1, p_dn_b1, p_dn_W2, p_dn_b2,
           p_cu_W1, p_cu_b1, p_cu_W2, p_cu_b2,
           p_cd_W1, p_cd_b1, p_cd_W2, p_cd_b2,
           p_cell_W1, p_cell_b1, p_cell_W2, p_cell_b2, p_cw,
           h, h_up, h_down, x, x_up, x_down,
           b_up_i, b_up_j, b_down_i, b_down_j):
    N, F = h.shape
    Nu, Fu = h_up.shape
    Nd, Fd = h_down.shape
    D = x.shape[1]
    H = p_up_b1.shape[1]
    O = p_cell_b2.shape[1]

    Fp = _ru(F, 128)
    Fm = max(_ru(Fu, 128), _ru(Fd, 128), Fp)
    Dp = _ru(D, 128)
    Op = _ru(O, 128)
    slab = H + Dp

    tP = min(512, _ru(max(N, Nu, Nd), 8))
    Np = _ru(max(N, Nu, Nd), tP)
    Eu, Ed = int(b_up_i.shape[0]), int(b_down_i.shape[0])
    tE = min(2048, _ru(max(Eu, Ed, 1), 8))
    E_pad = _ru(max(Eu, Ed, 1), tE)
    n_e = E_pad // tE
    vmem_lim = 48 * 2**20

    wia = jnp.pad(jnp.concatenate([p_up_W1[:F], p_dn_W1[:F]], axis=1),
                  ((0, Fm - F), (0, 0))).astype(_BF16)
    wj = jnp.concatenate(
        [jnp.pad(p_up_W1[F:F + Fu], ((0, Fm - Fu), (0, 0))),
         jnp.pad(p_dn_W1[F:F + Fd], ((0, Fm - Fd), (0, 0)))],
        axis=0).astype(_BF16)
    bj = jnp.concatenate([p_up_b1, p_dn_b1], axis=1)

    wm = jnp.stack([
        jnp.concatenate([p_up_W2, p_cu_W1, jnp.tile(p_cu_W2, (1, H))], 0),
        jnp.concatenate([p_dn_W2, p_cd_W1, jnp.tile(p_cd_W2, (1, H))], 0),
    ]).astype(_BF16)
    bv = jnp.stack([
        jnp.concatenate([p_up_W1[F + Fu:F + Fu + 1], p_up_b2, p_cu_b1,
                         jnp.tile(p_cu_b2, (1, H))], 0),
        jnp.concatenate([p_dn_W1[F + Fd:F + Fd + 1], p_dn_b2, p_cd_b1,
                         jnp.tile(p_cd_b2, (1, H))], 0),
    ])

    hall = jnp.stack([jnp.pad(h, ((0, Np - N), (0, Fm - F))),
                      jnp.pad(h_up, ((0, Np - Nu), (0, Fm - Fu))),
                      jnp.pad(h_down, ((0, Np - Nd), (0, Fm - Fd)))]
                     ).astype(_BF16)
    xall = jnp.stack([jnp.pad(x, ((0, Np - N), (0, Dp - D))),
                      jnp.pad(x_up, ((0, Np - Nu), (0, Dp - D))),
                      jnp.pad(x_down, ((0, Np - Nd), (0, Dp - D)))]
                     ).astype(_F32)

    proj = pl.pallas_call(
        _proj_kernel,
        grid=(Np // tP,),
        in_specs=[
            pl.BlockSpec((None, tP, Fm), lambda i: (0, i, 0)),
            pl.BlockSpec((None, tP, Fm), lambda i: (1, i, 0)),
            pl.BlockSpec((None, tP, Fm), lambda i: (2, i, 0)),
            pl.BlockSpec((None, tP, Dp), lambda i: (0, i, 0)),
            pl.BlockSpec((None, tP, Dp), lambda i: (1, i, 0)),
            pl.BlockSpec((None, tP, Dp), lambda i: (2, i, 0)),
            pl.BlockSpec((Fm, 2 * H), lambda i: (0, 0)),
            pl.BlockSpec((2 * Fm, H), lambda i: (0, 0)),
            pl.BlockSpec((1, 2 * H), lambda i: (0, 0)),
        ],
        out_specs=[pl.BlockSpec((2, tP, 2 * H), lambda i: (0, i, 0)),
                   pl.BlockSpec((2, 2 * tP, Dp), lambda i: (0, i, 0))],
        out_shape=[jax.ShapeDtypeStruct((2, Np, 2 * H), _BF16),
                   jax.ShapeDtypeStruct((2, 2 * Np, Dp), _F32)],
        compiler_params=pltpu.CompilerParams(
            dimension_semantics=("parallel",), vmem_limit_bytes=vmem_lim),
    )
    ti_all, tj_all = proj(hall, hall, hall, xall, xall, xall, wia, wj, bj)

    sbi_u, sbj_u = jax.lax.sort_key_val(b_up_i.astype(jnp.int32),
                                        b_up_j.astype(jnp.int32))
    sbi_d, sbj_d = jax.lax.sort_key_val(b_down_i.astype(jnp.int32),
                                        b_down_j.astype(jnp.int32))

    def gpad(idx, E):
        return jnp.pad(idx * 2, (0, E_pad - E))

    def spad(idx, E):
        return jnp.pad(idx, (0, E_pad - E), constant_values=Np)

    gj = jnp.stack([gpad(sbj_u, Eu),
                    gpad(sbj_d, Ed)]).reshape(2, 1, E_pad)
    siv = jnp.stack([spad(sbi_u, Eu),
                     spad(sbi_d, Ed)]).reshape(2, 1, E_pad)

    W = min(512, Np)
    keys = siv.reshape(2, E_pad)
    kc = jnp.minimum(keys, Np - 1)
    win = jnp.stack([kc[:, 0::tE] // W, kc[:, tE - 1::tE] // W], axis=1)

    smem = pltpu.MemorySpace.SMEM
    edge = pl.pallas_call(
        _make_edge_kernel(tE),
        grid=(2, n_e),
        in_specs=[
            pl.BlockSpec((None, 1, tE), lambda b, e: (b, 0, e),
                         memory_space=smem),
            pl.BlockSpec((None, 1, tE), lambda b, e: (b, 0, e)),
            pl.BlockSpec((None, 2, n_e), lambda b, e: (b, 0, 0),
                         memory_space=smem),
            pl.BlockSpec((None, Np, 2 * H), lambda b, e: (b, 0, 0)),
            pl.BlockSpec((None, 2 * Np, Dp), lambda b, e: (b, 0, 0)),
            pl.BlockSpec((None, 3 * H, H), lambda b, e: (b, 0, 0)),
            pl.BlockSpec((None, 4, H), lambda b, e: (b, 0, 0)),
        ],
        out_specs=pl.BlockSpec((Np, slab), lambda b, e: (0, b)),
        out_shape=jax.ShapeDtypeStruct((Np, 2 * slab), _F32),
        scratch_shapes=[pltpu.VMEM(((tE + 1) * 2, Dp), _F32),
                        pltpu.VMEM((tE, 2 * H), _F32)],
        compiler_params=pltpu.CompilerParams(
            dimension_semantics=("parallel", "arbitrary"),
            vmem_limit_bytes=vmem_lim),
    )
    agg = edge(gj, siv, win, ti_all, tj_all, wm, bv)

    cin = jnp.concatenate(
        [jnp.pad(h.astype(_F32), ((0, Np - N), (0, Fp - F))),
         jnp.pad(x.astype(_F32), ((0, Np - N), (0, Dp - D)))], axis=-1)

    w1c = jnp.concatenate(
        [jnp.pad(p_cell_W1[:F], ((0, Fp - F), (0, 0))),
         p_cell_W1[F:F + 2 * H]], axis=0).astype(_BF16)
    w2c = jnp.pad(p_cell_W2, ((0, 0), (0, Op - O))).astype(_BF16)
    b2c = jnp.pad(p_cell_b2, ((0, 0), (0, Op - O)))
    cw = p_cw.reshape(-1).astype(_F32)

    tN = min(512, _ru(N, 8))
    cell = pl.pallas_call(
        _cell_kernel,
        grid=(Np // tN,),
        in_specs=[
            pl.BlockSpec(memory_space=smem),
            pl.BlockSpec((tN, Fp + Dp), lambda i: (i, 0)),
            pl.BlockSpec((tN, 2 * slab), lambda i: (i, 0)),
            pl.BlockSpec((Fp + 2 * H, H), lambda i: (0, 0)),
            pl.BlockSpec((1, H), lambda i: (0, 0)),
            pl.BlockSpec((H, Op), lambda i: (0, 0)),
            pl.BlockSpec((1, Op), lambda i: (0, 0)),
        ],
        out_specs=pl.BlockSpec((tN, Op + Dp), lambda i: (i, 0)),
        out_shape=jax.ShapeDtypeStruct((Np, Op + Dp), _F32),
        compiler_params=pltpu.CompilerParams(
            dimension_semantics=("parallel",), vmem_limit_bytes=vmem_lim),
    )
    out = cell(cw, cin, agg, w1c, p_cell_b1, w2c, b2c)

    return out[:N, :O], out[:N, Op:Op + D]
